# baseline (speedup 1.0000x reference)
import numpy as np
import jax
import jax.numpy as jnp
from jax import lax
from jax.experimental import pallas as pl
from jax.experimental.pallas import tpu as pltpu

B = 128
S = 25
N = S + 1
NN = N * N
P_PIX = B * NN
EMBD = 128
LABD = 5
NK = 13
KP = 32
PPE = NK * KP
PB = B * PPE
EC = 8
CH = EC * PPE
NCHUNK = B // EC
NB = 32
GRID = B // NB
F_OUT = (192, 192, 96, 96)

_f32 = jnp.float32
_bf16 = jnp.bfloat16


def _pair_index_tables():
    idx = np.zeros((N, N), np.int32)
    for i in range(N):
        for j in range(N):
            if i == j:
                idx[i, j] = N
                continue
            k = (j - i) % N
            if k <= NK:
                idx[i, j] = (k - 1) * KP + i
            else:
                k2 = (i - j) % N
                idx[i, j] = (k2 - 1) * KP + j
    w = np.zeros((PPE, 1), np.float32)
    diag_budget = 2 * N
    for kb in range(NK):
        w[kb * KP:kb * KP + N, 0] = 2.0 if kb < NK - 1 else 1.0
        take = min(KP - N, diag_budget)
        w[kb * KP + N:kb * KP + N + take, 0] = 0.5
        diag_budget -= take
    return idx.reshape(-1), np.tile(w, (EC, 1)).reshape(1, -1)


_PAIR_IDX, _ROW_W = _pair_index_tables()


def _lrelu(v):
    return jnp.maximum(v, 0.01 * v)


def _bn_fold_vec(stats, g, bt):
    mean = stats[0:1, :] / float(P_PIX)
    var = stats[1:2, :] / float(P_PIX) - mean * mean
    s = g * lax.rsqrt(var + 1e-5)
    return s, bt - mean * s


def _wstats(hb, rwT):
    ps = jnp.dot(rwT, hb, preferred_element_type=_f32)
    pss = jnp.dot(rwT, hb * hb, preferred_element_type=_f32)
    return jnp.concatenate([ps, pss], axis=0)


def _make_round_kernel(r, d, d_prev):
    def body(refs):
        if r == 0:
            (z_ref, zi_ref, lab_ref, rw_ref, w0_ref, b0_ref, w1_ref, b1_ref,
             w2_ref, b2_ref, w3_ref, b3_ref, g01_ref, bt01_ref, g2_ref,
             bt2_ref, g3_ref, bt3_ref, wl_ref, x_ref, l_ref,
             hbuf, qbuf) = refs
            feats = jnp.concatenate([z_ref[...][:, None, :], zi_ref[...]],
                                    axis=1)
            labs = jnp.concatenate(
                [jnp.zeros((B, 1, LABD), _f32), lab_ref[...]], axis=1)
            x_ref[...] = jnp.concatenate([feats, labs], axis=2)
        else:
            (xp_ref, y_ref, yst_ref, gg_ref, gbt_ref, rw_ref, w0_ref, b0_ref,
             w1_ref, b1_ref, w2_ref, b2_ref, w3_ref, b3_ref, g01_ref,
             bt01_ref, g2_ref, bt2_ref, g3_ref, bt3_ref, wl_ref, x_ref,
             l_ref, hbuf, qbuf) = refs
            mean = yst_ref[0:1, :] / float(B * N)
            var = yst_ref[1:2, :] / float(B * N) - mean * mean
            sy = gg_ref[...] * lax.rsqrt(var + 1e-5)
            ty = gbt_ref[...] - mean * sy
            xn = _lrelu(y_ref[...] * sy[None, :, :].reshape(1, 1, 48)
                        + ty[None, :, :].reshape(1, 1, 48))
            x_ref[...] = jnp.concatenate([xp_ref[...], xn], axis=2)

        rw = rw_ref[...].astype(_bf16)

        def l0_body(i, st):
            xs = x_ref[pl.ds(i * EC, EC), :, :].astype(_bf16)
            zpad = jnp.zeros((EC, KP - N, d), _bf16)
            xp = jnp.concatenate([xs, zpad], axis=1)
            blocks = []
            for k in range(1, NK + 1):
                sh = jnp.concatenate([xs[:, k:, :], xs[:, :k, :], zpad],
                                     axis=1)
                blocks.append(jnp.abs(xp - sh))
            a = jnp.concatenate(blocks, axis=1).reshape(CH, d)
            h = jnp.dot(a, w0_ref[...], preferred_element_type=_f32) \
                + b0_ref[...]
            hb = h.astype(_bf16)
            hbuf[pl.ds(i * CH, CH), :] = hb
            return st + _wstats(hb, rw)

        st0 = lax.fori_loop(0, NCHUNK, l0_body,
                            jnp.zeros((2, F_OUT[0]), _f32))
        s0, t0 = _bn_fold_vec(st0, g01_ref[0:1, :], bt01_ref[0:1, :])
        s0 = s0.astype(_bf16)
        t0 = t0.astype(_bf16)

        def l1_body(i, st):
            hp = hbuf[pl.ds(i * CH, CH), :]
            a = _lrelu(hp * s0 + t0)
            h = jnp.dot(a, w1_ref[...], preferred_element_type=_f32) \
                + b1_ref[...]
            hb = h.astype(_bf16)
            hbuf[pl.ds(i * CH, CH), :] = hb
            return st + _wstats(hb, rw)

        st1 = lax.fori_loop(0, NCHUNK, l1_body,
                            jnp.zeros((2, F_OUT[1]), _f32))
        s1, t1 = _bn_fold_vec(st1, g01_ref[1:2, :], bt01_ref[1:2, :])
        s1 = s1.astype(_bf16)
        t1 = t1.astype(_bf16)

        def l2_body(i, st):
            hp = hbuf[pl.ds(i * CH, CH), :]
            a = _lrelu(hp * s1 + t1)
            h = jnp.dot(a, w2_ref[...], preferred_element_type=_f32) \
                + b2_ref[...]
            hb = h.astype(_bf16)
            qbuf[pl.ds(i * CH, CH), :] = hb
            return st + _wstats(hb, rw)

        st2 = lax.fori_loop(0, NCHUNK, l2_body,
                            jnp.zeros((2, F_OUT[2]), _f32))
        s2, t2 = _bn_fold_vec(st2, g2_ref[...], bt2_ref[...])
        s2 = s2.astype(_bf16)
        t2 = t2.astype(_bf16)

        def l3_body(i, st):
            hp = qbuf[pl.ds(i * CH, CH), :]
            a = _lrelu(hp * s2 + t2)
            h = jnp.dot(a, w3_ref[...], preferred_element_type=_f32) \
                + b3_ref[...]
            hb = h.astype(_bf16)
            qbuf[pl.ds(i * CH, CH), :] = hb
            return st + _wstats(hb, rw)

        st3 = lax.fori_loop(0, NCHUNK, l3_body,
                            jnp.zeros((2, F_OUT[3]), _f32))
        s3, t3 = _bn_fold_vec(st3, g3_ref[...], bt3_ref[...])
        s3 = s3.astype(_bf16)
        t3 = t3.astype(_bf16)

        def lp_body(i, _):
            hp = qbuf[pl.ds(i * CH, CH), :]
            a = _lrelu(hp * s3 + t3)
            l = jnp.dot(a, wl_ref[...], preferred_element_type=_f32)
            l_ref[pl.ds(i * EC, EC), :] = l.reshape(EC, PPE)
            return _

        lax.fori_loop(0, NCHUNK, lp_body, 0)

    def kern(*refs):
        body(refs)

    return kern


def _passE_kernel(lm_ref, x_ref, w1_ref, w2_ref, bg_ref, y_ref, stats_ref):
    ib = pl.program_id(0)
    logit = lm_ref[...]
    ii = lax.broadcasted_iota(jnp.int32, logit.shape, 1)
    jj = lax.broadcasted_iota(jnp.int32, logit.shape, 2)
    logit = logit - jnp.where(ii == jj, 1e8, 0.0)
    logit = logit - jnp.max(logit, axis=-1, keepdims=True)
    e = jnp.exp(logit)
    adj = e / jnp.sum(e, axis=-1, keepdims=True)
    xb = x_ref[...]
    d = xb.shape[-1]
    agg = lax.dot_general(adj, xb, (((2,), (1,)), ((0,), (0,))),
                          preferred_element_type=_f32)
    y = (jnp.dot(xb.reshape(NB * N, d), w1_ref[...],
                 preferred_element_type=_f32)
         + jnp.dot(agg.reshape(NB * N, d), w2_ref[...],
                   preferred_element_type=_f32)
         + bg_ref[...])
    ps = jnp.sum(y, axis=0)[None, :]
    pss = jnp.sum(y * y, axis=0)[None, :]
    part = jnp.concatenate([ps, pss], axis=0)

    @pl.when(ib == 0)
    def _():
        stats_ref[...] = part

    @pl.when(ib > 0)
    def _():
        stats_ref[...] += part

    y_ref[...] = y.reshape(NB, N, 48)


def _passEf_kernel(l0_ref, x_ref, w1_ref, w2_ref, bg_ref, sig_ref, ls_ref):
    logit = l0_ref[...]
    jj = lax.broadcasted_iota(jnp.int32, logit.shape, 1)
    logit = logit - jnp.where(jj == 0, 1e8, 0.0)
    logit = logit - jnp.max(logit, axis=-1, keepdims=True)
    e = jnp.exp(logit)
    adj0 = (e / jnp.sum(e, axis=-1, keepdims=True))[:, None, :]
    xb = x_ref[...]
    d = xb.shape[-1]
    agg = lax.dot_general(adj0, xb, (((2,), (1,)), ((0,), (0,))),
                          preferred_element_type=_f32)
    lg = (jnp.dot(xb[:, 0, :], w1_ref[...], preferred_element_type=_f32)
          + jnp.dot(agg.reshape(NB, d), w2_ref[...],
                    preferred_element_type=_f32)
          + bg_ref[...])
    sig_ref[...] = 1.0 / (1.0 + jnp.exp(-lg))
    m = jnp.max(lg, axis=1, keepdims=True)
    ls_ref[...] = lg - (m + jnp.log(jnp.sum(jnp.exp(lg - m), axis=1,
                                            keepdims=True)))


def _seq_params():
    return pltpu.CompilerParams(dimension_semantics=("arbitrary",))


def _full_spec(shape):
    return pl.BlockSpec(shape, lambda ib: tuple(0 for _ in shape))


def _blk_spec(shape):
    return pl.BlockSpec(shape, lambda ib: (ib,) + tuple(0 for _ in shape[1:]))


def _gc_weights(gp, d):
    w1 = gp['fc_w'][:, :d].T
    w2 = gp['fc_w'][:, d:].T
    return w1, w2, gp['fc_b'][None, :]


def kernel(z, zi_s, labels_yi, params):
    zi_t = jnp.transpose(zi_s, (1, 0, 2))
    lab_t = jnp.transpose(labels_yi, (1, 0, 2))
    row_w = jnp.asarray(_ROW_W)
    pair_idx = jnp.asarray(_PAIR_IDX)

    dims = (EMBD + LABD, EMBD + LABD + 48, EMBD + LABD + 96)
    x = None
    y_raw = None
    y_stats = None
    for r in range(3):
        wp = params['wc%d' % r] if r < 2 else params['wcl']
        gp = params['gc%d' % r] if r < 2 else params['gcl']
        d = dims[r]
        d_prev = dims[r - 1] if r else None
        wrefs = [wp['w0'].T.astype(_bf16), wp['b0'][None, :],
                 wp['w1'].T.astype(_bf16), wp['b1'][None, :],
                 wp['w2'].T.astype(_bf16), wp['b2'][None, :],
                 wp['w3'].T.astype(_bf16), wp['b3'][None, :],
                 jnp.stack([wp['g0'], wp['g1']]),
                 jnp.stack([wp['bt0'], wp['bt1']]),
                 wp['g2'][None, :], wp['bt2'][None, :],
                 wp['g3'][None, :], wp['bt3'][None, :],
                 wp['wl'].T.astype(_bf16)]
        wspecs = [_full_spec(tuple(a.shape)) for a in wrefs]
        scratch = [pltpu.VMEM((PB, 192), _bf16), pltpu.VMEM((PB, 96), _bf16)]
        outs = [jax.ShapeDtypeStruct((B, N, d), _f32),
                jax.ShapeDtypeStruct((B, PPE), _f32)]
        ospecs = [_full_spec((B, N, d)), _full_spec((B, PPE))]
        if r == 0:
            ins = [z, zi_t, lab_t, row_w] + wrefs
            ispecs = [_full_spec((B, EMBD)), _full_spec((B, S, EMBD)),
                      _full_spec((B, S, LABD)), _full_spec((1, CH))] + wspecs
        else:
            ins = [x, y_raw, y_stats, gp_prev['g'][None, :],
                   gp_prev['bt'][None, :], row_w] + wrefs
            ispecs = [_full_spec((B, N, d_prev)), _full_spec((B, N, 48)),
                      _full_spec((2, 48)), _full_spec((1, 48)),
                      _full_spec((1, 48)), _full_spec((1, CH))] + wspecs
        x, l_pk = pl.pallas_call(
            _make_round_kernel(r, d, d_prev),
            grid=(1,),
            in_specs=ispecs,
            out_specs=ospecs,
            out_shape=outs,
            scratch_shapes=scratch,
            compiler_params=_seq_params(),
        )(*ins)

        w1g, w2g, bg = _gc_weights(gp, d)
        if r < 2:
            l_mat = jnp.take(l_pk, pair_idx, axis=1).reshape(B, N, N)
            y_raw, y_stats = pl.pallas_call(
                _passE_kernel,
                grid=(GRID,),
                in_specs=[_blk_spec((NB, N, N)),
                          _blk_spec((NB, N, d)),
                          _full_spec((d, 48)),
                          _full_spec((d, 48)),
                          _full_spec((1, 48))],
                out_specs=[_blk_spec((NB, N, 48)),
                           _full_spec((2, 48))],
                out_shape=[jax.ShapeDtypeStruct((B, N, 48), _f32),
                           jax.ShapeDtypeStruct((2, 48), _f32)],
                compiler_params=_seq_params(),
            )(l_mat, x, w1g, w2g, bg)
            gp_prev = gp
        else:
            l0 = jnp.take(l_pk, pair_idx[:N], axis=1)
            sig, ls = pl.pallas_call(
                _passEf_kernel,
                grid=(GRID,),
                in_specs=[_blk_spec((NB, N)),
                          _blk_spec((NB, N, d)),
                          _full_spec((d, LABD)),
                          _full_spec((d, LABD)),
                          _full_spec((1, LABD))],
                out_specs=[_blk_spec((NB, LABD)),
                           _blk_spec((NB, LABD))],
                out_shape=[jax.ShapeDtypeStruct((B, LABD), _f32),
                           jax.ShapeDtypeStruct((B, LABD), _f32)],
                compiler_params=_seq_params(),
            )(l0, x, w1g, w2g, bg)
            return sig, ls


# speedup vs baseline: 1.2530x; 1.2530x over previous
import numpy as np
import jax
import jax.numpy as jnp
from jax import lax
from jax.experimental import pallas as pl
from jax.experimental.pallas import tpu as pltpu

B = 128
S = 25
N = S + 1
NN = N * N
P_PIX = B * NN
EMBD = 128
LABD = 5
NK = 13
KP = 32
PPE = NK * KP
PB = B * PPE
EC = 8
CH = EC * PPE
NCHUNK = B // EC
NB = 32
GRID = B // NB
F_OUT = (192, 192, 96, 96)

_f32 = jnp.float32
_bf16 = jnp.bfloat16


def _pair_index_tables():
    idx = np.zeros((N, N), np.int32)
    for i in range(N):
        for j in range(N):
            if i == j:
                idx[i, j] = N
                continue
            k = (j - i) % N
            if k <= NK:
                idx[i, j] = (k - 1) * KP + i
            else:
                k2 = (i - j) % N
                idx[i, j] = (k2 - 1) * KP + j
    w = np.zeros((PPE, 1), np.float32)
    diag_budget = 2 * N
    for kb in range(NK):
        w[kb * KP:kb * KP + N, 0] = 2.0 if kb < NK - 1 else 1.0
        take = min(KP - N, diag_budget)
        w[kb * KP + N:kb * KP + N + take, 0] = 0.5
        diag_budget -= take
    return idx.reshape(-1), np.tile(w, (EC, 1))


_PAIR_IDX, _ROW_W = _pair_index_tables()


def _lrelu(v):
    return jnp.maximum(v, 0.01 * v)


def _bn_fold_vec(stats, g, bt):
    mean = stats[0:1, :] / float(P_PIX)
    var = stats[1:2, :] / float(P_PIX) - mean * mean
    s = g * lax.rsqrt(var + 1e-5)
    return s, bt - mean * s


def _wstats(h, w):
    wh = h * w
    ps = jnp.sum(wh, axis=0)[None, :]
    pss = jnp.sum(wh * h, axis=0)[None, :]
    return jnp.concatenate([ps, pss], axis=0)


def _make_round_kernel(r, d, d_prev):
    def body(refs):
        if r == 0:
            (z_ref, zi_ref, lab_ref, rw_ref, w0_ref, b0_ref, w1_ref, b1_ref,
             w2_ref, b2_ref, w3_ref, b3_ref, g01_ref, bt01_ref, g2_ref,
             bt2_ref, g3_ref, bt3_ref, wl_ref, x_ref, l_ref,
             hbuf, qbuf) = refs
            feats = jnp.concatenate([z_ref[...][:, None, :], zi_ref[...]],
                                    axis=1)
            labs = jnp.concatenate(
                [jnp.zeros((B, 1, LABD), _f32), lab_ref[...]], axis=1)
            x_ref[...] = jnp.concatenate([feats, labs], axis=2)
        else:
            (xp_ref, y_ref, yst_ref, gg_ref, gbt_ref, rw_ref, w0_ref, b0_ref,
             w1_ref, b1_ref, w2_ref, b2_ref, w3_ref, b3_ref, g01_ref,
             bt01_ref, g2_ref, bt2_ref, g3_ref, bt3_ref, wl_ref, x_ref,
             l_ref, hbuf, qbuf) = refs
            mean = yst_ref[0:1, :] / float(B * N)
            var = yst_ref[1:2, :] / float(B * N) - mean * mean
            sy = gg_ref[...] * lax.rsqrt(var + 1e-5)
            ty = gbt_ref[...] - mean * sy
            xn = _lrelu(y_ref[...] * sy[None, :, :].reshape(1, 1, 48)
                        + ty[None, :, :].reshape(1, 1, 48))
            x_ref[...] = jnp.concatenate([xp_ref[...], xn], axis=2)

        rw = rw_ref[...]

        def l0_body(i, st):
            xs = x_ref[pl.ds(i * EC, EC), :, :].astype(_bf16)
            zpad = jnp.zeros((EC, KP - N, d), _bf16)
            xp = jnp.concatenate([xs, zpad], axis=1)
            blocks = []
            for k in range(1, NK + 1):
                sh = jnp.concatenate([xs[:, k:, :], xs[:, :k, :], zpad],
                                     axis=1)
                blocks.append(jnp.abs(xp - sh))
            a = jnp.concatenate(blocks, axis=1).reshape(CH, d)
            h = jnp.dot(a, w0_ref[...], preferred_element_type=_f32) \
                + b0_ref[...]
            hbuf[pl.ds(i * CH, CH), :] = h.astype(_bf16)
            return st + _wstats(h, rw)

        st0 = lax.fori_loop(0, NCHUNK, l0_body,
                            jnp.zeros((2, F_OUT[0]), _f32), unroll=2)
        s0, t0 = _bn_fold_vec(st0, g01_ref[0:1, :], bt01_ref[0:1, :])
        s0 = s0.astype(_bf16)
        t0 = t0.astype(_bf16)

        def l1_body(i, st):
            hp = hbuf[pl.ds(i * CH, CH), :]
            a = _lrelu(hp * s0 + t0)
            h = jnp.dot(a, w1_ref[...], preferred_element_type=_f32) \
                + b1_ref[...]
            hbuf[pl.ds(i * CH, CH), :] = h.astype(_bf16)
            return st + _wstats(h, rw)

        st1 = lax.fori_loop(0, NCHUNK, l1_body,
                            jnp.zeros((2, F_OUT[1]), _f32), unroll=2)
        s1, t1 = _bn_fold_vec(st1, g01_ref[1:2, :], bt01_ref[1:2, :])
        s1 = s1.astype(_bf16)
        t1 = t1.astype(_bf16)

        def l2_body(i, st):
            hp = hbuf[pl.ds(i * CH, CH), :]
            a = _lrelu(hp * s1 + t1)
            h = jnp.dot(a, w2_ref[...], preferred_element_type=_f32) \
                + b2_ref[...]
            qbuf[pl.ds(i * CH, CH), :] = h.astype(_bf16)
            return st + _wstats(h, rw)

        st2 = lax.fori_loop(0, NCHUNK, l2_body,
                            jnp.zeros((2, F_OUT[2]), _f32), unroll=2)
        s2, t2 = _bn_fold_vec(st2, g2_ref[...], bt2_ref[...])
        s2 = s2.astype(_bf16)
        t2 = t2.astype(_bf16)

        def l3_body(i, st):
            hp = qbuf[pl.ds(i * CH, CH), :]
            a = _lrelu(hp * s2 + t2)
            h = jnp.dot(a, w3_ref[...], preferred_element_type=_f32) \
                + b3_ref[...]
            qbuf[pl.ds(i * CH, CH), :] = h.astype(_bf16)
            return st + _wstats(h, rw)

        st3 = lax.fori_loop(0, NCHUNK, l3_body,
                            jnp.zeros((2, F_OUT[3]), _f32), unroll=2)
        s3, t3 = _bn_fold_vec(st3, g3_ref[...], bt3_ref[...])
        s3 = s3.astype(_bf16)
        t3 = t3.astype(_bf16)

        def lp_body(i, _):
            hp = qbuf[pl.ds(i * CH, CH), :]
            a = _lrelu(hp * s3 + t3)
            l = jnp.dot(a, wl_ref[...], preferred_element_type=_f32)
            l_ref[pl.ds(i * EC, EC), :] = l.reshape(EC, PPE)
            return _

        lax.fori_loop(0, NCHUNK, lp_body, 0, unroll=2)

    def kern(*refs):
        body(refs)

    return kern


def _passE_kernel(lm_ref, x_ref, w1_ref, w2_ref, bg_ref, y_ref, stats_ref):
    ib = pl.program_id(0)
    logit = lm_ref[...]
    ii = lax.broadcasted_iota(jnp.int32, logit.shape, 1)
    jj = lax.broadcasted_iota(jnp.int32, logit.shape, 2)
    logit = logit - jnp.where(ii == jj, 1e8, 0.0)
    logit = logit - jnp.max(logit, axis=-1, keepdims=True)
    e = jnp.exp(logit)
    adj = e / jnp.sum(e, axis=-1, keepdims=True)
    xb = x_ref[...]
    d = xb.shape[-1]
    agg = lax.dot_general(adj, xb, (((2,), (1,)), ((0,), (0,))),
                          preferred_element_type=_f32)
    y = (jnp.dot(xb.reshape(NB * N, d), w1_ref[...],
                 preferred_element_type=_f32)
         + jnp.dot(agg.reshape(NB * N, d), w2_ref[...],
                   preferred_element_type=_f32)
         + bg_ref[...])
    ps = jnp.sum(y, axis=0)[None, :]
    pss = jnp.sum(y * y, axis=0)[None, :]
    part = jnp.concatenate([ps, pss], axis=0)

    @pl.when(ib == 0)
    def _():
        stats_ref[...] = part

    @pl.when(ib > 0)
    def _():
        stats_ref[...] += part

    y_ref[...] = y.reshape(NB, N, 48)


def _passEf_kernel(l0_ref, x_ref, w1_ref, w2_ref, bg_ref, sig_ref, ls_ref):
    logit = l0_ref[...]
    jj = lax.broadcasted_iota(jnp.int32, logit.shape, 1)
    logit = logit - jnp.where(jj == 0, 1e8, 0.0)
    logit = logit - jnp.max(logit, axis=-1, keepdims=True)
    e = jnp.exp(logit)
    adj0 = (e / jnp.sum(e, axis=-1, keepdims=True))[:, None, :]
    xb = x_ref[...]
    d = xb.shape[-1]
    agg = lax.dot_general(adj0, xb, (((2,), (1,)), ((0,), (0,))),
                          preferred_element_type=_f32)
    lg = (jnp.dot(xb[:, 0, :], w1_ref[...], preferred_element_type=_f32)
          + jnp.dot(agg.reshape(NB, d), w2_ref[...],
                    preferred_element_type=_f32)
          + bg_ref[...])
    sig_ref[...] = 1.0 / (1.0 + jnp.exp(-lg))
    m = jnp.max(lg, axis=1, keepdims=True)
    ls_ref[...] = lg - (m + jnp.log(jnp.sum(jnp.exp(lg - m), axis=1,
                                            keepdims=True)))


def _seq_params():
    return pltpu.CompilerParams(dimension_semantics=("arbitrary",))


def _full_spec(shape):
    return pl.BlockSpec(shape, lambda ib: tuple(0 for _ in shape))


def _blk_spec(shape):
    return pl.BlockSpec(shape, lambda ib: (ib,) + tuple(0 for _ in shape[1:]))


def _gc_weights(gp, d):
    w1 = gp['fc_w'][:, :d].T
    w2 = gp['fc_w'][:, d:].T
    return w1, w2, gp['fc_b'][None, :]


def kernel(z, zi_s, labels_yi, params):
    zi_t = jnp.transpose(zi_s, (1, 0, 2))
    lab_t = jnp.transpose(labels_yi, (1, 0, 2))
    row_w = jnp.asarray(_ROW_W)
    pair_idx = jnp.asarray(_PAIR_IDX)

    dims = (EMBD + LABD, EMBD + LABD + 48, EMBD + LABD + 96)
    x = None
    y_raw = None
    y_stats = None
    for r in range(3):
        wp = params['wc%d' % r] if r < 2 else params['wcl']
        gp = params['gc%d' % r] if r < 2 else params['gcl']
        d = dims[r]
        d_prev = dims[r - 1] if r else None
        wrefs = [wp['w0'].T.astype(_bf16), wp['b0'][None, :],
                 wp['w1'].T.astype(_bf16), wp['b1'][None, :],
                 wp['w2'].T.astype(_bf16), wp['b2'][None, :],
                 wp['w3'].T.astype(_bf16), wp['b3'][None, :],
                 jnp.stack([wp['g0'], wp['g1']]),
                 jnp.stack([wp['bt0'], wp['bt1']]),
                 wp['g2'][None, :], wp['bt2'][None, :],
                 wp['g3'][None, :], wp['bt3'][None, :],
                 wp['wl'].T.astype(_bf16)]
        wspecs = [_full_spec(tuple(a.shape)) for a in wrefs]
        scratch = [pltpu.VMEM((PB, 192), _bf16), pltpu.VMEM((PB, 96), _bf16)]
        outs = [jax.ShapeDtypeStruct((B, N, d), _f32),
                jax.ShapeDtypeStruct((B, PPE), _f32)]
        ospecs = [_full_spec((B, N, d)), _full_spec((B, PPE))]
        if r == 0:
            ins = [z, zi_t, lab_t, row_w] + wrefs
            ispecs = [_full_spec((B, EMBD)), _full_spec((B, S, EMBD)),
                      _full_spec((B, S, LABD)), _full_spec((CH, 1))] + wspecs
        else:
            ins = [x, y_raw, y_stats, gp_prev['g'][None, :],
                   gp_prev['bt'][None, :], row_w] + wrefs
            ispecs = [_full_spec((B, N, d_prev)), _full_spec((B, N, 48)),
                      _full_spec((2, 48)), _full_spec((1, 48)),
                      _full_spec((1, 48)), _full_spec((CH, 1))] + wspecs
        x, l_pk = pl.pallas_call(
            _make_round_kernel(r, d, d_prev),
            grid=(1,),
            in_specs=ispecs,
            out_specs=ospecs,
            out_shape=outs,
            scratch_shapes=scratch,
            compiler_params=_seq_params(),
        )(*ins)

        w1g, w2g, bg = _gc_weights(gp, d)
        if r < 2:
            l_mat = jnp.take(l_pk, pair_idx, axis=1).reshape(B, N, N)
            y_raw, y_stats = pl.pallas_call(
                _passE_kernel,
                grid=(GRID,),
                in_specs=[_blk_spec((NB, N, N)),
                          _blk_spec((NB, N, d)),
                          _full_spec((d, 48)),
                          _full_spec((d, 48)),
                          _full_spec((1, 48))],
                out_specs=[_blk_spec((NB, N, 48)),
                           _full_spec((2, 48))],
                out_shape=[jax.ShapeDtypeStruct((B, N, 48), _f32),
                           jax.ShapeDtypeStruct((2, 48), _f32)],
                compiler_params=_seq_params(),
            )(l_mat, x, w1g, w2g, bg)
            gp_prev = gp
        else:
            l0 = jnp.take(l_pk, pair_idx[:N], axis=1)
            sig, ls = pl.pallas_call(
                _passEf_kernel,
                grid=(GRID,),
                in_specs=[_blk_spec((NB, N)),
                          _blk_spec((NB, N, d)),
                          _full_spec((d, LABD)),
                          _full_spec((d, LABD)),
                          _full_spec((1, LABD))],
                out_specs=[_blk_spec((NB, LABD)),
                           _blk_spec((NB, LABD))],
                out_shape=[jax.ShapeDtypeStruct((B, LABD), _f32),
                           jax.ShapeDtypeStruct((B, LABD), _f32)],
                compiler_params=_seq_params(),
            )(l0, x, w1g, w2g, bg)
            return sig, ls


# speedup vs baseline: 1.2697x; 1.0133x over previous
import numpy as np
import jax
import jax.numpy as jnp
from jax import lax
from jax.experimental import pallas as pl
from jax.experimental.pallas import tpu as pltpu

B = 128
S = 25
N = S + 1
NN = N * N
P_PIX = B * NN
EMBD = 128
LABD = 5
NK = 13
KP = 32
PPE = NK * KP
PB = B * PPE
EC = 8
CH = EC * PPE
NCHUNK = B // EC
NB = 32
GRID = B // NB
F_OUT = (192, 192, 96, 96)

_f32 = jnp.float32
_bf16 = jnp.bfloat16


def _pair_index_tables():
    idx = np.zeros((N, N), np.int32)
    for i in range(N):
        for j in range(N):
            if i == j:
                idx[i, j] = N
                continue
            k = (j - i) % N
            if k <= NK:
                idx[i, j] = (k - 1) * KP + i
            else:
                k2 = (i - j) % N
                idx[i, j] = (k2 - 1) * KP + j
    w = np.zeros((PPE, 1), np.float32)
    diag_budget = 2 * N
    for kb in range(NK):
        w[kb * KP:kb * KP + N, 0] = 2.0 if kb < NK - 1 else 1.0
        take = min(KP - N, diag_budget)
        w[kb * KP + N:kb * KP + N + take, 0] = 0.5
        diag_budget -= take
    return idx.reshape(-1), np.tile(w, (EC, 1))


_PAIR_IDX, _ROW_W = _pair_index_tables()


def _lrelu(v):
    return jnp.maximum(v, 0.01 * v)


def _bn_fold_vec(stats, g, bt):
    mean = stats[0:1, :] / float(P_PIX)
    var = stats[1:2, :] / float(P_PIX) - mean * mean
    s = g * lax.rsqrt(var + 1e-5)
    return s, bt - mean * s


def _wstats(h, w):
    wh = h * w
    ps = jnp.sum(wh, axis=0)[None, :]
    pss = jnp.sum(wh * h, axis=0)[None, :]
    return jnp.concatenate([ps, pss], axis=0)


def _make_round_kernel(r, d, d_prev):
    def body(refs):
        if r == 0:
            (z_ref, zi_ref, lab_ref, rw_ref, w0_ref, b0_ref, w1_ref, b1_ref,
             w2_ref, b2_ref, w3_ref, b3_ref, g01_ref, bt01_ref, g2_ref,
             bt2_ref, g3_ref, bt3_ref, wl_ref, x_ref, l_ref,
             hbuf, qbuf) = refs
            feats = jnp.concatenate([z_ref[...][:, None, :], zi_ref[...]],
                                    axis=1)
            labs = jnp.concatenate(
                [jnp.zeros((B, 1, LABD), _f32), lab_ref[...]], axis=1)
            x_ref[...] = jnp.concatenate([feats, labs], axis=2)
        else:
            (xp_ref, y_ref, yst_ref, gg_ref, gbt_ref, rw_ref, w0_ref, b0_ref,
             w1_ref, b1_ref, w2_ref, b2_ref, w3_ref, b3_ref, g01_ref,
             bt01_ref, g2_ref, bt2_ref, g3_ref, bt3_ref, wl_ref, x_ref,
             l_ref, hbuf, qbuf) = refs
            mean = yst_ref[0:1, :] / float(B * N)
            var = yst_ref[1:2, :] / float(B * N) - mean * mean
            sy = gg_ref[...] * lax.rsqrt(var + 1e-5)
            ty = gbt_ref[...] - mean * sy
            xn = _lrelu(y_ref[...] * sy[None, :, :].reshape(1, 1, 48)
                        + ty[None, :, :].reshape(1, 1, 48))
            x_ref[...] = jnp.concatenate([xp_ref[...], xn], axis=2)

        rw = rw_ref[...]

        def l0_body(i, st):
            xs = x_ref[pl.ds(i * EC, EC), :, :].astype(_bf16)
            zpad = jnp.zeros((EC, KP - N, d), _bf16)
            xp = jnp.concatenate([xs, zpad], axis=1)
            blocks = []
            for k in range(1, NK + 1):
                sh = jnp.concatenate([xs[:, k:, :], xs[:, :k, :], zpad],
                                     axis=1)
                blocks.append(jnp.abs(xp - sh))
            a = jnp.concatenate(blocks, axis=1).reshape(CH, d)
            h = jnp.dot(a, w0_ref[...], preferred_element_type=_f32) \
                + b0_ref[...]
            hbuf[pl.ds(i * CH, CH), :] = h.astype(_bf16)
            return st + _wstats(h, rw)

        st0 = lax.fori_loop(0, NCHUNK, l0_body,
                            jnp.zeros((2, F_OUT[0]), _f32), unroll=4)
        s0, t0 = _bn_fold_vec(st0, g01_ref[0:1, :], bt01_ref[0:1, :])
        s0 = s0.astype(_bf16)
        t0 = t0.astype(_bf16)

        def l1_body(i, st):
            hp = hbuf[pl.ds(i * CH, CH), :]
            a = _lrelu(hp * s0 + t0)
            h = jnp.dot(a, w1_ref[...], preferred_element_type=_f32) \
                + b1_ref[...]
            hbuf[pl.ds(i * CH, CH), :] = h.astype(_bf16)
            return st + _wstats(h, rw)

        st1 = lax.fori_loop(0, NCHUNK, l1_body,
                            jnp.zeros((2, F_OUT[1]), _f32), unroll=4)
        s1, t1 = _bn_fold_vec(st1, g01_ref[1:2, :], bt01_ref[1:2, :])
        s1 = s1.astype(_bf16)
        t1 = t1.astype(_bf16)

        def l2_body(i, st):
            hp = hbuf[pl.ds(i * CH, CH), :]
            a = _lrelu(hp * s1 + t1)
            h = jnp.dot(a, w2_ref[...], preferred_element_type=_f32) \
                + b2_ref[...]
            qbuf[pl.ds(i * CH, CH), :] = h.astype(_bf16)
            return st + _wstats(h, rw)

        st2 = lax.fori_loop(0, NCHUNK, l2_body,
                            jnp.zeros((2, F_OUT[2]), _f32), unroll=4)
        s2, t2 = _bn_fold_vec(st2, g2_ref[...], bt2_ref[...])
        s2 = s2.astype(_bf16)
        t2 = t2.astype(_bf16)

        def l3_body(i, st):
            hp = qbuf[pl.ds(i * CH, CH), :]
            a = _lrelu(hp * s2 + t2)
            h = jnp.dot(a, w3_ref[...], preferred_element_type=_f32) \
                + b3_ref[...]
            qbuf[pl.ds(i * CH, CH), :] = h.astype(_bf16)
            return st + _wstats(h, rw)

        st3 = lax.fori_loop(0, NCHUNK, l3_body,
                            jnp.zeros((2, F_OUT[3]), _f32), unroll=4)
        s3, t3 = _bn_fold_vec(st3, g3_ref[...], bt3_ref[...])
        s3 = s3.astype(_bf16)
        t3 = t3.astype(_bf16)

        def lp_body(i, _):
            hp = qbuf[pl.ds(i * CH, CH), :]
            a = _lrelu(hp * s3 + t3)
            l = jnp.dot(a, wl_ref[...], preferred_element_type=_f32)
            l_ref[pl.ds(i * EC, EC), :] = l.reshape(EC, PPE)
            return _

        lax.fori_loop(0, NCHUNK, lp_body, 0, unroll=4)

    def kern(*refs):
        body(refs)

    return kern


def _passE_kernel(lm_ref, x_ref, w1_ref, w2_ref, bg_ref, y_ref, stats_ref):
    ib = pl.program_id(0)
    logit = lm_ref[...]
    ii = lax.broadcasted_iota(jnp.int32, logit.shape, 1)
    jj = lax.broadcasted_iota(jnp.int32, logit.shape, 2)
    logit = logit - jnp.where(ii == jj, 1e8, 0.0)
    logit = logit - jnp.max(logit, axis=-1, keepdims=True)
    e = jnp.exp(logit)
    adj = e / jnp.sum(e, axis=-1, keepdims=True)
    xb = x_ref[...]
    d = xb.shape[-1]
    agg = lax.dot_general(adj, xb, (((2,), (1,)), ((0,), (0,))),
                          preferred_element_type=_f32)
    y = (jnp.dot(xb.reshape(NB * N, d), w1_ref[...],
                 preferred_element_type=_f32)
         + jnp.dot(agg.reshape(NB * N, d), w2_ref[...],
                   preferred_element_type=_f32)
         + bg_ref[...])
    ps = jnp.sum(y, axis=0)[None, :]
    pss = jnp.sum(y * y, axis=0)[None, :]
    part = jnp.concatenate([ps, pss], axis=0)

    @pl.when(ib == 0)
    def _():
        stats_ref[...] = part

    @pl.when(ib > 0)
    def _():
        stats_ref[...] += part

    y_ref[...] = y.reshape(NB, N, 48)


def _passEf_kernel(l0_ref, x_ref, w1_ref, w2_ref, bg_ref, sig_ref, ls_ref):
    logit = l0_ref[...]
    jj = lax.broadcasted_iota(jnp.int32, logit.shape, 1)
    logit = logit - jnp.where(jj == 0, 1e8, 0.0)
    logit = logit - jnp.max(logit, axis=-1, keepdims=True)
    e = jnp.exp(logit)
    adj0 = (e / jnp.sum(e, axis=-1, keepdims=True))[:, None, :]
    xb = x_ref[...]
    d = xb.shape[-1]
    agg = lax.dot_general(adj0, xb, (((2,), (1,)), ((0,), (0,))),
                          preferred_element_type=_f32)
    lg = (jnp.dot(xb[:, 0, :], w1_ref[...], preferred_element_type=_f32)
          + jnp.dot(agg.reshape(NB, d), w2_ref[...],
                    preferred_element_type=_f32)
          + bg_ref[...])
    sig_ref[...] = 1.0 / (1.0 + jnp.exp(-lg))
    m = jnp.max(lg, axis=1, keepdims=True)
    ls_ref[...] = lg - (m + jnp.log(jnp.sum(jnp.exp(lg - m), axis=1,
                                            keepdims=True)))


def _seq_params():
    return pltpu.CompilerParams(dimension_semantics=("arbitrary",))


def _full_spec(shape):
    return pl.BlockSpec(shape, lambda ib: tuple(0 for _ in shape))


def _blk_spec(shape):
    return pl.BlockSpec(shape, lambda ib: (ib,) + tuple(0 for _ in shape[1:]))


def _gc_weights(gp, d):
    w1 = gp['fc_w'][:, :d].T
    w2 = gp['fc_w'][:, d:].T
    return w1, w2, gp['fc_b'][None, :]


def kernel(z, zi_s, labels_yi, params):
    zi_t = jnp.transpose(zi_s, (1, 0, 2))
    lab_t = jnp.transpose(labels_yi, (1, 0, 2))
    row_w = jnp.asarray(_ROW_W)
    pair_idx = jnp.asarray(_PAIR_IDX)

    dims = (EMBD + LABD, EMBD + LABD + 48, EMBD + LABD + 96)
    x = None
    y_raw = None
    y_stats = None
    for r in range(3):
        wp = params['wc%d' % r] if r < 2 else params['wcl']
        gp = params['gc%d' % r] if r < 2 else params['gcl']
        d = dims[r]
        d_prev = dims[r - 1] if r else None
        wrefs = [wp['w0'].T.astype(_bf16), wp['b0'][None, :],
                 wp['w1'].T.astype(_bf16), wp['b1'][None, :],
                 wp['w2'].T.astype(_bf16), wp['b2'][None, :],
                 wp['w3'].T.astype(_bf16), wp['b3'][None, :],
                 jnp.stack([wp['g0'], wp['g1']]),
                 jnp.stack([wp['bt0'], wp['bt1']]),
                 wp['g2'][None, :], wp['bt2'][None, :],
                 wp['g3'][None, :], wp['bt3'][None, :],
                 wp['wl'].T.astype(_bf16)]
        wspecs = [_full_spec(tuple(a.shape)) for a in wrefs]
        scratch = [pltpu.VMEM((PB, 192), _bf16), pltpu.VMEM((PB, 96), _bf16)]
        outs = [jax.ShapeDtypeStruct((B, N, d), _f32),
                jax.ShapeDtypeStruct((B, PPE), _f32)]
        ospecs = [_full_spec((B, N, d)), _full_spec((B, PPE))]
        if r == 0:
            ins = [z, zi_t, lab_t, row_w] + wrefs
            ispecs = [_full_spec((B, EMBD)), _full_spec((B, S, EMBD)),
                      _full_spec((B, S, LABD)), _full_spec((CH, 1))] + wspecs
        else:
            ins = [x, y_raw, y_stats, gp_prev['g'][None, :],
                   gp_prev['bt'][None, :], row_w] + wrefs
            ispecs = [_full_spec((B, N, d_prev)), _full_spec((B, N, 48)),
                      _full_spec((2, 48)), _full_spec((1, 48)),
                      _full_spec((1, 48)), _full_spec((CH, 1))] + wspecs
        x, l_pk = pl.pallas_call(
            _make_round_kernel(r, d, d_prev),
            grid=(1,),
            in_specs=ispecs,
            out_specs=ospecs,
            out_shape=outs,
            scratch_shapes=scratch,
            compiler_params=_seq_params(),
        )(*ins)

        w1g, w2g, bg = _gc_weights(gp, d)
        if r < 2:
            l_mat = jnp.take(l_pk, pair_idx, axis=1).reshape(B, N, N)
            y_raw, y_stats = pl.pallas_call(
                _passE_kernel,
                grid=(GRID,),
                in_specs=[_blk_spec((NB, N, N)),
                          _blk_spec((NB, N, d)),
                          _full_spec((d, 48)),
                          _full_spec((d, 48)),
                          _full_spec((1, 48))],
                out_specs=[_blk_spec((NB, N, 48)),
                           _full_spec((2, 48))],
                out_shape=[jax.ShapeDtypeStruct((B, N, 48), _f32),
                           jax.ShapeDtypeStruct((2, 48), _f32)],
                compiler_params=_seq_params(),
            )(l_mat, x, w1g, w2g, bg)
            gp_prev = gp
        else:
            l0 = jnp.take(l_pk, pair_idx[:N], axis=1)
            sig, ls = pl.pallas_call(
                _passEf_kernel,
                grid=(GRID,),
                in_specs=[_blk_spec((NB, N)),
                          _blk_spec((NB, N, d)),
                          _full_spec((d, LABD)),
                          _full_spec((d, LABD)),
                          _full_spec((1, LABD))],
                out_specs=[_blk_spec((NB, LABD)),
                           _blk_spec((NB, LABD))],
                out_shape=[jax.ShapeDtypeStruct((B, LABD), _f32),
                           jax.ShapeDtypeStruct((B, LABD), _f32)],
                compiler_params=_seq_params(),
            )(l0, x, w1g, w2g, bg)
            return sig, ls
